# Initial kernel scaffold; baseline (speedup 1.0000x reference)
import jax
import jax.numpy as jnp
from jax.experimental import pallas as pl


def kernel(x, router_w, w1, w2):
    raise NotImplementedError("write your pallas kernel here")


# speedup vs baseline: 3.4352x; 3.4352x over previous
import jax
import jax.numpy as jnp
from jax.experimental import pallas as pl
from jax.experimental.pallas import tpu as pltpu

NUM_EXPERTS = 16
TOP_K = 8
N_EMBD = 768
D_FFN = 384


def _moe_kernel(x_ref, rw_ref, w1_ref, w2_ref, out_ref, comb_ref, acc_ref):
    e = pl.program_id(0)

    @pl.when(e == 0)
    def _routing():
        xt = x_ref[...]
        logits = jax.lax.dot_general(
            xt, rw_ref[...], (((1,), (1,)), ((), ())),
            preferred_element_type=jnp.float32)
        m = jnp.max(logits, axis=-1, keepdims=True)
        p = jnp.exp(logits - m)
        p = p / jnp.sum(p, axis=-1, keepdims=True)
        T = p.shape[0]
        col = jax.lax.broadcasted_iota(jnp.int32, (T, NUM_EXPERTS), 1)
        rank = jnp.zeros((T, NUM_EXPERTS), dtype=jnp.int32)
        for j in range(NUM_EXPERTS):
            pj = p[:, j:j + 1]
            beats = (pj > p) | ((pj == p) & (col > j))
            rank = rank + beats.astype(jnp.int32)
        w = jnp.where(rank < TOP_K, p, 0.0)
        w = w / jnp.sum(w, axis=-1, keepdims=True)
        comb_ref[...] = w
        acc_ref[...] = jnp.zeros_like(acc_ref)

    x = x_ref[...]
    h = jax.lax.dot_general(
        x, w1_ref[...], (((1,), (0,)), ((), ())),
        preferred_element_type=jnp.float32)
    h = jax.nn.gelu(h)
    y = jax.lax.dot_general(
        h, w2_ref[...], (((1,), (0,)), ((), ())),
        preferred_element_type=jnp.float32)
    comb = comb_ref[...]
    col = jax.lax.broadcasted_iota(jnp.int32, comb.shape, 1)
    ce = jnp.sum(jnp.where(col == e, comb, 0.0), axis=1, keepdims=True)
    acc_ref[...] += ce * y

    @pl.when(e == NUM_EXPERTS - 1)
    def _finish():
        out_ref[...] = acc_ref[...]


def kernel(x, router_w, w1, w2):
    B, S, D = x.shape
    T = B * S
    xt = x.reshape(T, D)
    out = pl.pallas_call(
        _moe_kernel,
        grid=(NUM_EXPERTS,),
        in_specs=[
            pl.BlockSpec((T, D), lambda e: (0, 0)),
            pl.BlockSpec((NUM_EXPERTS, D), lambda e: (0, 0)),
            pl.BlockSpec((D, D_FFN), lambda e: (0, e)),
            pl.BlockSpec((D_FFN, D), lambda e: (e, 0)),
        ],
        out_specs=pl.BlockSpec((T, D), lambda e: (0, 0)),
        out_shape=jax.ShapeDtypeStruct((T, D), jnp.float32),
        scratch_shapes=[
            pltpu.VMEM((T, NUM_EXPERTS), jnp.float32),
            pltpu.VMEM((T, D), jnp.float32),
        ],
        compiler_params=pltpu.CompilerParams(
            dimension_semantics=("arbitrary",),
        ),
    )(xt, router_w, w1, w2)
    return out.reshape(B, S, D)
